# baseline (speedup 1.0000x reference)
import jax
import jax.numpy as jnp
from jax import lax
from jax.experimental import pallas as pl
from jax.experimental.pallas import tpu as pltpu
from jax.experimental.pallas import tpu_sc as plsc

VOCAB = 1000000
DIM = 64
SEQ = 50
BATCH = 16384

NC = 2
NS = 16
NW = NC * NS
LANES = 16

B_PER_W = BATCH // NW
GATHER_CHUNK = 128
N_GATHER = B_PER_W // GATHER_CHUNK
DT = DIM // 8
TB_PER_W = B_PER_W // 128
BLK = TB_PER_W * 8 * 128

FMT_BV = 4096
HALF_BLOCKS = 123
HALF = HALF_BLOCKS * FMT_BV


def _fmt_kernel(a_ref, b_ref, p_ref):
    p_ref[...] = jnp.concatenate(
        [jnp.transpose(a_ref[...]), jnp.transpose(b_ref[...])], axis=1)


def _table_rowmajor(table_t):
    p = pl.pallas_call(
        _fmt_kernel,
        grid=(HALF_BLOCKS,),
        in_specs=[
            pl.BlockSpec((DIM, FMT_BV), lambda i: (0, i)),
            pl.BlockSpec((DIM, FMT_BV),
                         lambda i: (0, jnp.minimum(i + HALF_BLOCKS,
                                                   VOCAB // FMT_BV))),
        ],
        out_specs=pl.BlockSpec((FMT_BV, 2 * DIM), lambda i: (i, 0)),
        out_shape=jax.ShapeDtypeStruct((HALF, 2 * DIM), jnp.float32),
    )(table_t, table_t)
    return p.reshape(2 * HALF, DIM)


def _sc_kernel(xt_hbm, tab_hbm, pos_hbm, out_hbm, idx_v, rows_v, blk_v,
               pos_v, sem):
    wid = lax.axis_index("s") * NC + lax.axis_index("c")
    b0 = wid * B_PER_W
    tb0 = wid * TB_PER_W

    pltpu.sync_copy(pos_hbm, pos_v)

    lane = lax.broadcasted_iota(jnp.int32, (LANES,), 0)
    cj = []
    for j in range(DIM // LANES):
        d = j * LANES + lane
        cj.append((d // 8) * BLK + (d % 8) * 128)

    def s_body(s, _):
        pltpu.sync_copy(xt_hbm.at[s, pl.ds(b0, B_PER_W)], idx_v)
        cps = []
        for k in range(N_GATHER):
            cps.append(pltpu.async_copy(
                tab_hbm.at[idx_v.at[pl.ds(k * GATHER_CHUNK, GATHER_CHUNK)]],
                rows_v.at[pl.ds(k * GATHER_CHUNK, GATHER_CHUNK)], sem))
        for cp in cps:
            cp.wait()

        pv = [pos_v[s, pl.ds(j * LANES, LANES)] for j in range(DIM // LANES)]

        def row_body(row, _):
            scal = (row >> 7) * 1024 + (row & 127)
            for j in range(DIM // LANES):
                vals = rows_v[row, pl.ds(j * LANES, LANES)] + pv[j]
                plsc.store_scatter(blk_v, [cj[j] + scal], vals)
            return 0

        lax.fori_loop(0, B_PER_W, row_body, 0)

        for td in range(DT):
            out_off = ((s * DT + td) * (BATCH // 128) + tb0) * 1024
            pltpu.sync_copy(blk_v.at[pl.ds(td * BLK, BLK)],
                            out_hbm.at[pl.ds(out_off, BLK)])
        return 0

    lax.fori_loop(0, SEQ, s_body, 0)


def kernel(x, token_table, pos_table):
    xt = x.T.astype(jnp.int32)
    xt = jnp.where(xt >= HALF, 2 * xt - (2 * HALF - 1), 2 * xt)
    tab = _table_rowmajor(token_table.T)
    mesh = plsc.VectorSubcoreMesh(core_axis_name="c", subcore_axis_name="s")
    k = pl.kernel(
        _sc_kernel,
        mesh=mesh,
        compiler_params=pltpu.CompilerParams(use_tc_tiling_on_sc=False,
                                             needs_layout_passes=False),
        out_type=jax.ShapeDtypeStruct((SEQ * DIM * BATCH,), jnp.float32),
        scratch_types=[
            pltpu.VMEM((B_PER_W,), jnp.int32),
            pltpu.VMEM((B_PER_W, DIM), jnp.float32),
            pltpu.VMEM((DT * BLK,), jnp.float32),
            pltpu.VMEM((SEQ, DIM), jnp.float32),
            pltpu.SemaphoreType.DMA,
        ],
    )
    out_flat = k(xt, tab, pos_table)
    out5 = out_flat.reshape(SEQ, DT, BATCH // 128, 8, 128)
    return out5.transpose(2, 4, 0, 1, 3).reshape(BATCH, SEQ, DIM)
